# baseline (speedup 1.0000x reference)
import functools

import jax
import jax.numpy as jnp
from jax import lax
from jax.experimental import pallas as pl
from jax.experimental.pallas import tpu as pltpu
from jax.experimental.pallas import tpu_sc as plsc

N = 4096
F = 512
K = 1024
INT_MIN = -2147483648


def _col_cumsum(x):
    row = jax.lax.broadcasted_iota(jnp.int32, x.shape, 0)
    s = 1
    while s < x.shape[0]:
        r = pltpu.roll(x, s, 0)
        x = x + jnp.where(row >= s, r, jnp.zeros_like(x))
        s *= 2
    return x


def _mega_kernel(x_ref, w_ref, s_ref, idx_ref, spool_ref, feat_ref):
    w = w_ref[...]
    rn = jax.lax.rsqrt(jnp.maximum(jnp.sum(w * w), 1e-12))
    x = x_ref[...]
    y = jax.lax.dot_general(
        x, w * rn, (((1,), (0,)), ((), ())),
        preferred_element_type=jnp.float32,
    )

    feat_ref[...] = x * jnp.tanh(y)

    b = jax.lax.bitcast_convert_type(y, jnp.int32)
    skey = b ^ (jax.lax.shift_right_arithmetic(b, 31) & jnp.int32(0x7FFFFFFF))

    def body(i, p):
        bit = jax.lax.shift_left(jnp.int32(1), 31 - i)
        cand = p | bit
        cand_s = cand ^ INT_MIN
        cnt = jnp.sum(jnp.where(skey >= cand_s, 1, 0))
        return jax.lax.select(cnt >= K, cand, p)

    p = jax.lax.fori_loop(0, 32, body, jnp.int32(0))
    t_s = p ^ INT_MIN

    gt = skey > t_s
    eq = skey == t_s
    n_gt = jnp.sum(jnp.where(gt, 1, 0))
    csum_eq = _col_cumsum(jnp.where(eq, 1, 0))
    mask = gt | (eq & (csum_eq <= (K - n_gt)))

    pos = _col_cumsum(jnp.where(mask, 1, 0)) - 1
    iota_f = jax.lax.broadcasted_iota(jnp.int32, (N, 1), 0).astype(jnp.float32)
    s_col = s_ref[...]
    CH = 128

    def chunk(c, _):
        pl_ = jax.lax.broadcasted_iota(jnp.int32, (N, CH), 1) + c * CH
        selc = jnp.where((pos == pl_) & mask, 1.0, 0.0)
        idx_c = jax.lax.dot_general(
            selc, iota_f, (((0,), (0,)), ((), ())),
            preferred_element_type=jnp.float32,
            precision=jax.lax.Precision.HIGHEST,
        )
        sp_c = jax.lax.dot_general(
            selc, s_col, (((0,), (0,)), ((), ())),
            preferred_element_type=jnp.float32,
            precision=jax.lax.Precision.HIGHEST,
        )
        idx_ref[pl.ds(c * CH, CH), :] = (idx_c + 0.5).astype(jnp.int32)
        spool_ref[pl.ds(c * CH, CH), :] = (sp_c + 0.5).astype(jnp.int32)
        return 0

    jax.lax.fori_loop(0, K // CH, chunk, 0)


def _transpose_kernel(a_ref, o_ref):
    o_ref[...] = a_ref[...].T


def _sc_gather_rows(table, idx, D, CH):
    BPW = K // 32
    mesh = plsc.VectorSubcoreMesh(core_axis_name="c", subcore_axis_name="s")

    @functools.partial(
        pl.kernel,
        mesh=mesh,
        out_type=jax.ShapeDtypeStruct((K, D), jnp.float32),
        scratch_types=[
            pltpu.VMEM((BPW,), jnp.int32),
            pltpu.VMEM((CH, D), jnp.float32),
            pltpu.SemaphoreType.DMA,
        ],
    )
    def gk(table_hbm, idx_hbm, out_hbm, idx_v, rows_v, sem):
        wid = lax.axis_index("s") * 2 + lax.axis_index("c")
        base = wid * BPW
        pltpu.sync_copy(idx_hbm.at[pl.ds(base, BPW)], idx_v)
        for c in range(BPW // CH):
            pltpu.async_copy(
                table_hbm.at[idx_v.at[pl.ds(c * CH, CH)]], rows_v, sem
            ).wait()
            pltpu.sync_copy(rows_v, out_hbm.at[pl.ds(base + c * CH, CH)])

    return gk(table, idx)


def _matmul_kernel(ar_ref, act_ref, o_ref):
    @pl.when(pl.program_id(0) == 0)
    def _():
        o_ref[...] = jnp.zeros_like(o_ref)

    o_ref[...] += jax.lax.dot_general(
        ar_ref[...], act_ref[...], (((1,), (1,)), ((), ())),
        preferred_element_type=jnp.float32,
    )


def kernel(X, A, S, kernel):
    s_col = S.astype(jnp.float32).reshape(N, 1)

    idx2, s_pool2, feat = pl.pallas_call(
        _mega_kernel,
        out_shape=(
            jax.ShapeDtypeStruct((K, 1), jnp.int32),
            jax.ShapeDtypeStruct((K, 1), jnp.int32),
            jax.ShapeDtypeStruct((N, F), jnp.float32),
        ),
    )(X, kernel, s_col)
    idx = idx2.reshape(K)
    S_pooled = s_pool2.reshape(K).astype(S.dtype)

    At = pl.pallas_call(
        _transpose_kernel,
        grid=(16, 16),
        in_specs=[pl.BlockSpec((256, 256), lambda i, j: (i, j))],
        out_specs=pl.BlockSpec((256, 256), lambda i, j: (j, i)),
        out_shape=jax.ShapeDtypeStruct((N, N), jnp.float32),
    )(A)

    Ar = _sc_gather_rows(A, idx, N, 8)
    Act = _sc_gather_rows(At, idx, N, 8)

    A_pooled = pl.pallas_call(
        _matmul_kernel,
        grid=(8,),
        in_specs=[
            pl.BlockSpec((K, N // 8), lambda k: (0, k)),
            pl.BlockSpec((K, N // 8), lambda k: (0, k)),
        ],
        out_specs=pl.BlockSpec((K, K), lambda k: (0, 0)),
        out_shape=jax.ShapeDtypeStruct((K, K), jnp.float32),
    )(Ar, Act)

    X_pooled = _sc_gather_rows(feat, idx, F, 32)

    return X_pooled, A_pooled, S_pooled


# speedup vs baseline: 1.1679x; 1.1679x over previous
import functools

import jax
import jax.numpy as jnp
from jax import lax
from jax.experimental import pallas as pl
from jax.experimental.pallas import tpu as pltpu
from jax.experimental.pallas import tpu_sc as plsc

N = 4096
F = 512
K = 1024
INT_MIN = -2147483648


def _lane_cumsum(x):
    lane = jax.lax.broadcasted_iota(jnp.int32, x.shape, 1)
    s = 1
    while s < x.shape[1]:
        r = pltpu.roll(x, s, 1)
        x = x + jnp.where(lane >= s, r, jnp.zeros_like(x))
        s *= 2
    return x


def _score_kernel(xt_ref, w_ref, y_ref):
    w = w_ref[...]
    rn = jax.lax.rsqrt(jnp.maximum(jnp.sum(w * w), 1e-12))
    y_ref[...] = jax.lax.dot_general(
        w * rn, xt_ref[...], (((1,), (0,)), ((), ())),
        preferred_element_type=jnp.float32,
    )


def _topk_kernel(y_ref, s_ref, idx_ref, spool_ref):
    yt = y_ref[...]
    b = jax.lax.bitcast_convert_type(yt, jnp.int32)
    skey = b ^ (jax.lax.shift_right_arithmetic(b, 31) & jnp.int32(0x7FFFFFFF))
    u = skey ^ INT_MIN

    binrow = jax.lax.broadcasted_iota(jnp.int32, (256, N), 0)
    active = jnp.ones_like(u, dtype=jnp.bool_)
    gtmask = jnp.zeros_like(active)
    kcur = jnp.int32(K)
    for sh in (24, 16, 8, 0):
        byte = jax.lax.shift_right_logical(u, sh) & 255
        ge_hits = active & (byte >= binrow)
        cnts_ge = jnp.sum(ge_hits.astype(jnp.int32), axis=1, keepdims=True)
        bidx = jax.lax.broadcasted_iota(jnp.int32, (256, 1), 0)
        bstar = jnp.max(jnp.where(cnts_ge >= kcur, bidx, -1))
        gt_here = active & (byte > bstar)
        kcur = kcur - jnp.sum(jnp.where(gt_here, 1, 0))
        gtmask = gtmask | gt_here
        active = active & (byte == bstar)

    csum_eq = _lane_cumsum(jnp.where(active, 1, 0))
    mask = gtmask | (active & (csum_eq <= kcur))

    pos = _lane_cumsum(jnp.where(mask, 1, 0)) - 1
    lane_i = jax.lax.broadcasted_iota(jnp.int32, (1, N), 1)
    s_row = s_ref[...]
    CH = 128

    def chunk(c, _):
        prow = jax.lax.broadcasted_iota(jnp.int32, (CH, N), 0) + c * CH
        selc = (pos == prow) & mask
        idx_c = jnp.max(jnp.where(selc, lane_i, -1), axis=1, keepdims=True)
        sp_c = jnp.max(jnp.where(selc, s_row, -1), axis=1, keepdims=True)
        idx_ref[pl.ds(c * CH, CH), :] = idx_c
        spool_ref[pl.ds(c * CH, CH), :] = sp_c
        return 0

    jax.lax.fori_loop(0, K // CH, chunk, 0)


def _feat_kernel(x_ref, y_ref, o_ref):
    o_ref[...] = x_ref[...] * jnp.tanh(y_ref[...])


def _transpose_kernel(a_ref, o_ref):
    o_ref[...] = a_ref[...].T


def _sc_gather_rows(table, idx, D, CH):
    BPW = K // 32
    mesh = plsc.VectorSubcoreMesh(core_axis_name="c", subcore_axis_name="s")

    @functools.partial(
        pl.kernel,
        mesh=mesh,
        out_type=jax.ShapeDtypeStruct((K, D), jnp.float32),
        scratch_types=[
            pltpu.VMEM((BPW,), jnp.int32),
            pltpu.VMEM((CH, D), jnp.float32),
            pltpu.SemaphoreType.DMA,
        ],
    )
    def gk(table_hbm, idx_hbm, out_hbm, idx_v, rows_v, sem):
        wid = lax.axis_index("s") * 2 + lax.axis_index("c")
        base = wid * BPW
        pltpu.sync_copy(idx_hbm.at[pl.ds(base, BPW)], idx_v)
        for c in range(BPW // CH):
            pltpu.async_copy(
                table_hbm.at[idx_v.at[pl.ds(c * CH, CH)]], rows_v, sem
            ).wait()
            pltpu.sync_copy(rows_v, out_hbm.at[pl.ds(base + c * CH, CH)])

    return gk(table, idx)


def _matmul_kernel(ar_ref, act_ref, o_ref):
    @pl.when(pl.program_id(0) == 0)
    def _():
        o_ref[...] = jnp.zeros_like(o_ref)

    o_ref[...] += jax.lax.dot_general(
        ar_ref[...], act_ref[...], (((1,), (1,)), ((), ())),
        preferred_element_type=jnp.float32,
    )


def kernel(X, A, S, kernel):
    XT = X.T
    w_row = kernel.reshape(1, F)
    s_row = S.reshape(1, N)

    yt = pl.pallas_call(
        _score_kernel,
        grid=(4,),
        in_specs=[
            pl.BlockSpec((F, N // 4), lambda i: (0, i)),
            pl.BlockSpec((1, F), lambda i: (0, 0)),
        ],
        out_specs=pl.BlockSpec((1, N // 4), lambda i: (0, i)),
        out_shape=jax.ShapeDtypeStruct((1, N), jnp.float32),
    )(XT, w_row)

    idx2, s_pool2 = pl.pallas_call(
        _topk_kernel,
        out_shape=(
            jax.ShapeDtypeStruct((K, 1), jnp.int32),
            jax.ShapeDtypeStruct((K, 1), jnp.int32),
        ),
    )(yt, s_row)
    idx = idx2.reshape(K)
    S_pooled = s_pool2.reshape(K).astype(S.dtype)

    At = pl.pallas_call(
        _transpose_kernel,
        grid=(16, 16),
        in_specs=[pl.BlockSpec((256, 256), lambda i, j: (i, j))],
        out_specs=pl.BlockSpec((256, 256), lambda i, j: (j, i)),
        out_shape=jax.ShapeDtypeStruct((N, N), jnp.float32),
    )(A)

    Ar = _sc_gather_rows(A, idx, N, 8)
    Act = _sc_gather_rows(At, idx, N, 8)

    A_pooled = pl.pallas_call(
        _matmul_kernel,
        grid=(8,),
        in_specs=[
            pl.BlockSpec((K, N // 8), lambda k: (0, k)),
            pl.BlockSpec((K, N // 8), lambda k: (0, k)),
        ],
        out_specs=pl.BlockSpec((K, K), lambda k: (0, 0)),
        out_shape=jax.ShapeDtypeStruct((K, K), jnp.float32),
    )(Ar, Act)

    y_col = yt.reshape(N, 1)
    feat = pl.pallas_call(
        _feat_kernel,
        grid=(16,),
        in_specs=[
            pl.BlockSpec((N // 16, F), lambda i: (i, 0)),
            pl.BlockSpec((N // 16, 1), lambda i: (i, 0)),
        ],
        out_specs=pl.BlockSpec((N // 16, F), lambda i: (i, 0)),
        out_shape=jax.ShapeDtypeStruct((N, F), jnp.float32),
    )(X, y_col)
    X_pooled = _sc_gather_rows(feat, idx, F, 32)

    return X_pooled, A_pooled, S_pooled


# speedup vs baseline: 1.8193x; 1.5578x over previous
import functools

import jax
import jax.numpy as jnp
from jax import lax
from jax.experimental import pallas as pl
from jax.experimental.pallas import tpu as pltpu
from jax.experimental.pallas import tpu_sc as plsc

N = 4096
F = 512
K = 1024
INT_MIN = -2147483648


def _lane_cumsum(x):
    lane = jax.lax.broadcasted_iota(jnp.int32, x.shape, 1)
    s = 1
    while s < x.shape[1]:
        r = pltpu.roll(x, s, 1)
        x = x + jnp.where(lane >= s, r, jnp.zeros_like(x))
        s *= 2
    return x


def _score_kernel(xt_ref, w_ref, y_ref):
    w = w_ref[...]
    rn = jax.lax.rsqrt(jnp.maximum(jnp.sum(w * w), 1e-12))
    y_ref[...] = jax.lax.dot_general(
        w * rn, xt_ref[...], (((1,), (0,)), ((), ())),
        preferred_element_type=jnp.float32,
    )


def _topk_kernel(y_ref, s_ref, idx_ref, spool_ref):
    yt = y_ref[...]
    b = jax.lax.bitcast_convert_type(yt, jnp.int32)
    skey = b ^ (jax.lax.shift_right_arithmetic(b, 31) & jnp.int32(0x7FFFFFFF))
    u = skey ^ INT_MIN

    binrow = jax.lax.broadcasted_iota(jnp.int32, (256, N), 0)
    active = jnp.ones_like(u, dtype=jnp.bool_)
    gtmask = jnp.zeros_like(active)
    kcur = jnp.int32(K)
    for sh in (24, 16, 8, 0):
        byte = jax.lax.shift_right_logical(u, sh) & 255
        ge_hits = active & (byte >= binrow)
        cnts_ge = jnp.sum(ge_hits.astype(jnp.int32), axis=1, keepdims=True)
        bidx = jax.lax.broadcasted_iota(jnp.int32, (256, 1), 0)
        bstar = jnp.max(jnp.where(cnts_ge >= kcur, bidx, -1))
        gt_here = active & (byte > bstar)
        kcur = kcur - jnp.sum(jnp.where(gt_here, 1, 0))
        gtmask = gtmask | gt_here
        active = active & (byte == bstar)

    csum_eq = _lane_cumsum(jnp.where(active, 1, 0))
    mask = gtmask | (active & (csum_eq <= kcur))

    pos = _lane_cumsum(jnp.where(mask, 1, 0)) - 1
    lane_i = jax.lax.broadcasted_iota(jnp.int32, (1, N), 1)
    s_row = s_ref[...]
    CH = 128

    def chunk(c, _):
        prow = jax.lax.broadcasted_iota(jnp.int32, (CH, N), 0) + c * CH
        selc = (pos == prow) & mask
        idx_c = jnp.max(jnp.where(selc, lane_i, -1), axis=1, keepdims=True)
        sp_c = jnp.max(jnp.where(selc, s_row, -1), axis=1, keepdims=True)
        idx_ref[pl.ds(c * CH, CH), :] = idx_c
        spool_ref[pl.ds(c * CH, CH), :] = sp_c
        return 0

    jax.lax.fori_loop(0, K // CH, chunk, 0)


def _feat_kernel(x_ref, y_ref, o_ref):
    o_ref[...] = x_ref[...] * jnp.tanh(y_ref[...])


def _transpose_kernel(a_ref, o_ref):
    o_ref[...] = a_ref[...].T


def _sc_gather_rows(table, idx, D, CH):
    BPW = K // 32
    mesh = plsc.VectorSubcoreMesh(core_axis_name="c", subcore_axis_name="s")

    @functools.partial(
        pl.kernel,
        mesh=mesh,
        out_type=jax.ShapeDtypeStruct((K, D), jnp.float32),
        scratch_types=[
            pltpu.VMEM((BPW,), jnp.int32),
            pltpu.VMEM((CH, D), jnp.float32),
            pltpu.SemaphoreType.DMA,
        ],
    )
    def gk(table_hbm, idx_hbm, out_hbm, idx_v, rows_v, sem):
        wid = lax.axis_index("s") * 2 + lax.axis_index("c")
        base = wid * BPW
        pltpu.sync_copy(idx_hbm.at[pl.ds(base, BPW)], idx_v)
        for c in range(BPW // CH):
            pltpu.async_copy(
                table_hbm.at[idx_v.at[pl.ds(c * CH, CH)]], rows_v, sem
            ).wait()
            pltpu.sync_copy(rows_v, out_hbm.at[pl.ds(base + c * CH, CH)])

    return gk(table, idx)


def _mt_kernel(a_ref, ar_ref, o_ref):
    @pl.when(pl.program_id(1) == 0)
    def _():
        o_ref[...] = jnp.zeros_like(o_ref)

    o_ref[...] += jax.lax.dot_general(
        a_ref[...], ar_ref[...], (((0,), (1,)), ((), ())),
        preferred_element_type=jnp.float32,
    )


def kernel(X, A, S, kernel):
    XT = X.T
    w_row = kernel.reshape(1, F)
    s_row = S.reshape(1, N)

    yt = pl.pallas_call(
        _score_kernel,
        grid=(4,),
        in_specs=[
            pl.BlockSpec((F, N // 4), lambda i: (0, i)),
            pl.BlockSpec((1, F), lambda i: (0, 0)),
        ],
        out_specs=pl.BlockSpec((1, N // 4), lambda i: (0, i)),
        out_shape=jax.ShapeDtypeStruct((1, N), jnp.float32),
    )(XT, w_row)

    idx2, s_pool2 = pl.pallas_call(
        _topk_kernel,
        out_shape=(
            jax.ShapeDtypeStruct((K, 1), jnp.int32),
            jax.ShapeDtypeStruct((K, 1), jnp.int32),
        ),
    )(yt, s_row)
    idx = idx2.reshape(K)
    S_pooled = s_pool2.reshape(K).astype(S.dtype)

    Ar = _sc_gather_rows(A, idx, N, 8)

    Mt = pl.pallas_call(
        _mt_kernel,
        grid=(8, 4),
        in_specs=[
            pl.BlockSpec((N // 4, N // 8), lambda c, k: (k, c)),
            pl.BlockSpec((K, N // 4), lambda c, k: (0, k)),
        ],
        out_specs=pl.BlockSpec((N // 8, K), lambda c, k: (c, 0)),
        out_shape=jax.ShapeDtypeStruct((N, K), jnp.float32),
    )(A, Ar)

    ApT = _sc_gather_rows(Mt, idx, K, 32)

    A_pooled = pl.pallas_call(
        _transpose_kernel,
        grid=(4, 4),
        in_specs=[pl.BlockSpec((256, 256), lambda i, j: (i, j))],
        out_specs=pl.BlockSpec((256, 256), lambda i, j: (j, i)),
        out_shape=jax.ShapeDtypeStruct((K, K), jnp.float32),
    )(ApT)

    y_col = yt.reshape(N, 1)
    feat = pl.pallas_call(
        _feat_kernel,
        grid=(16,),
        in_specs=[
            pl.BlockSpec((N // 16, F), lambda i: (i, 0)),
            pl.BlockSpec((N // 16, 1), lambda i: (i, 0)),
        ],
        out_specs=pl.BlockSpec((N // 16, F), lambda i: (i, 0)),
        out_shape=jax.ShapeDtypeStruct((N, F), jnp.float32),
    )(X, y_col)
    X_pooled = _sc_gather_rows(feat, idx, F, 32)

    return X_pooled, A_pooled, S_pooled
